# baseline (device time: 99409 ns/iter reference)
import functools

import jax
import jax.numpy as jnp
from jax import lax
from jax.experimental import pallas as pl
from jax.experimental.pallas import tpu as pltpu

N_DEV = 4
KTILE = 1024
N_A = 4
N_CHUNK = 4
CHUNK = 512
HALF = 256


def _layer_body(x_ref, win_ref, wout_ref, out_ref,
                h_ref, comm_ref, send_sems, recv_sems):
    t = pl.program_id(0)
    my_pos = lax.axis_index("i")
    left = (my_pos + N_DEV - 1) % N_DEV
    right = (my_pos + 1) % N_DEV
    p_xor = my_pos ^ 1
    p_rev = 3 - my_pos

    @pl.when(t < N_A)
    def _():
        h_ref[:, pl.ds(t * KTILE, KTILE)] = jnp.maximum(
            jnp.dot(x_ref[...], win_ref[...],
                    preferred_element_type=jnp.float32),
            0.0,
        )

    @pl.when(t == 0)
    def _():
        barrier = pltpu.get_barrier_semaphore()
        for nbr in (left, right):
            pl.semaphore_signal(
                barrier, inc=1,
                device_id=(nbr,), device_id_type=pl.DeviceIdType.MESH,
            )

    @pl.when(t >= N_A)
    def _():
        c = t - N_A
        out_ref[:, pl.ds(c * CHUNK, CHUNK)] = jnp.dot(
            h_ref[...], wout_ref[...],
            preferred_element_type=jnp.float32,
        )

    def rdmas(round_, c_):
        slot_a = 4 * c_ + 2 * round_
        slot_b = slot_a + 1
        pa = p_xor if round_ == 0 else p_rev
        pb = p_rev if round_ == 0 else p_xor
        a = pltpu.make_async_remote_copy(
            src_ref=out_ref.at[:, pl.ds(c_ * CHUNK, HALF)],
            dst_ref=comm_ref.at[slot_a],
            send_sem=send_sems.at[slot_a],
            recv_sem=recv_sems.at[slot_a],
            device_id=(pa,),
            device_id_type=pl.DeviceIdType.MESH,
        )
        b = pltpu.make_async_remote_copy(
            src_ref=out_ref.at[:, pl.ds(c_ * CHUNK + HALF, HALF)],
            dst_ref=comm_ref.at[slot_b],
            send_sem=send_sems.at[slot_b],
            recv_sem=recv_sems.at[slot_b],
            device_id=(pb,),
            device_id_type=pl.DeviceIdType.MESH,
        )
        return a, b

    def start(round_, c_):
        a, b = rdmas(round_, c_)
        a.start()
        b.start()

    def wait_and_add(round_, c_):
        a, b = rdmas(round_, c_)
        a.wait()
        b.wait()
        slot_a = 4 * c_ + 2 * round_
        out_ref[:, pl.ds(c_ * CHUNK, HALF)] += comm_ref[slot_a, :, :]
        out_ref[:, pl.ds(c_ * CHUNK + HALF, HALF)] += comm_ref[slot_a + 1, :, :]

    for c in range(N_CHUNK):
        @pl.when(t == N_A + c)
        def _(c=c):
            if c == 0:
                pl.semaphore_wait(pltpu.get_barrier_semaphore(), 2)
            start(0, c)
            if c >= 1:
                wait_and_add(0, c - 1)
                start(1, c - 1)
            if c >= 2:
                wait_and_add(1, c - 2)
            if c == N_CHUNK - 1:
                wait_and_add(0, c)
                start(1, c)
                wait_and_add(1, c - 1)
                wait_and_add(1, c)


def _layer(x, win, wout, cid):
    m, d_in = x.shape
    d_hid = win.shape[1]
    d_out = wout.shape[1]
    n_steps = N_A + N_CHUNK

    def win_idx(j):
        return (0, jnp.clip(j, 0, N_A - 1))

    def wout_idx(j):
        return (0, jnp.clip(j - N_A, 0, N_CHUNK - 1))

    return pl.pallas_call(
        functools.partial(_layer_body),
        grid=(n_steps,),
        out_shape=jax.ShapeDtypeStruct((m, d_out), jnp.float32),
        in_specs=[
            pl.BlockSpec((m, d_in), lambda j: (0, 0)),
            pl.BlockSpec((d_in, KTILE), win_idx),
            pl.BlockSpec((d_hid, CHUNK), wout_idx),
        ],
        out_specs=pl.BlockSpec((m, d_out), lambda j: (0, 0)),
        scratch_shapes=[
            pltpu.VMEM((m, d_hid), jnp.float32),
            pltpu.VMEM((16, m, HALF), jnp.float32),
            pltpu.SemaphoreType.DMA((16,)),
            pltpu.SemaphoreType.DMA((16,)),
        ],
        compiler_params=pltpu.CompilerParams(
            dimension_semantics=("arbitrary",),
            collective_id=cid,
            vmem_limit_bytes=56 * 1024 * 1024,
        ),
    )(x, win, wout)


def kernel(x, Win0, Wout0, Win1, Wout1, Win2, Wout2):
    x = _layer(x, Win0, Wout0, 0)
    x = _layer(x, Win1, Wout1, 1)
    x = _layer(x, Win2, Wout2, 2)
    return x


# device time: 93897 ns/iter; 1.0587x vs baseline; 1.0587x over previous
import functools

import jax
import jax.numpy as jnp
from jax import lax
from jax.experimental import pallas as pl
from jax.experimental.pallas import tpu as pltpu

N_DEV = 4
KTILE = 512
N_A = 8
N_CHUNK = 4
CHUNK = 512
HALF = 256


def _layer_body(x_ref, win_ref, wout_ref, out_ref,
                h_ref, comm_ref, send_sems, recv_sems):
    t = pl.program_id(0)
    my_pos = lax.axis_index("i")
    left = (my_pos + N_DEV - 1) % N_DEV
    right = (my_pos + 1) % N_DEV
    p_xor = my_pos ^ 1
    p_rev = 3 - my_pos

    @pl.when(t < N_A)
    def _():
        h_ref[:, pl.ds(t * KTILE, KTILE)] = jnp.maximum(
            jnp.dot(x_ref[...], win_ref[...],
                    preferred_element_type=jnp.float32),
            0.0,
        )

    @pl.when(t >= N_A)
    def _():
        c = t - N_A
        out_ref[:, pl.ds(c * CHUNK, CHUNK)] = jnp.dot(
            h_ref[...], wout_ref[...],
            preferred_element_type=jnp.float32,
        )

    def rdmas(round_, c_):
        slot_a = 4 * c_ + 2 * round_
        slot_b = slot_a + 1
        pa = p_xor if round_ == 0 else p_rev
        pb = p_rev if round_ == 0 else p_xor
        a = pltpu.make_async_remote_copy(
            src_ref=out_ref.at[:, pl.ds(c_ * CHUNK, HALF)],
            dst_ref=comm_ref.at[slot_a],
            send_sem=send_sems.at[slot_a],
            recv_sem=recv_sems.at[slot_a],
            device_id=(pa,),
            device_id_type=pl.DeviceIdType.MESH,
        )
        b = pltpu.make_async_remote_copy(
            src_ref=out_ref.at[:, pl.ds(c_ * CHUNK + HALF, HALF)],
            dst_ref=comm_ref.at[slot_b],
            send_sem=send_sems.at[slot_b],
            recv_sem=recv_sems.at[slot_b],
            device_id=(pb,),
            device_id_type=pl.DeviceIdType.MESH,
        )
        return a, b

    def start(round_, c_):
        a, b = rdmas(round_, c_)
        a.start()
        b.start()

    def wait_and_add(round_, c_):
        a, b = rdmas(round_, c_)
        a.wait()
        b.wait()
        slot_a = 4 * c_ + 2 * round_
        out_ref[:, pl.ds(c_ * CHUNK, HALF)] += comm_ref[slot_a, :, :]
        out_ref[:, pl.ds(c_ * CHUNK + HALF, HALF)] += comm_ref[slot_a + 1, :, :]

    for c in range(N_CHUNK):
        @pl.when(t == N_A + c)
        def _(c=c):
            if c == 0:
                barrier = pltpu.get_barrier_semaphore()
                for nbr in (left, right):
                    pl.semaphore_signal(
                        barrier, inc=1,
                        device_id=(nbr,), device_id_type=pl.DeviceIdType.MESH,
                    )
                pl.semaphore_wait(barrier, 2)
            start(0, c)
            if c >= 1:
                wait_and_add(0, c - 1)
                start(1, c - 1)
            if c >= 2:
                wait_and_add(1, c - 2)
            if c == N_CHUNK - 1:
                wait_and_add(0, c)
                start(1, c)
                wait_and_add(1, c - 1)
                wait_and_add(1, c)


def _layer(x, win, wout, cid):
    m, d_in = x.shape
    d_hid = win.shape[1]
    d_out = wout.shape[1]
    n_steps = N_A + N_CHUNK

    def win_idx(j):
        return (0, jnp.clip(j, 0, N_A - 1))

    def wout_idx(j):
        return (0, jnp.clip(j - N_A, 0, N_CHUNK - 1))

    return pl.pallas_call(
        functools.partial(_layer_body),
        grid=(n_steps,),
        out_shape=jax.ShapeDtypeStruct((m, d_out), jnp.float32),
        in_specs=[
            pl.BlockSpec((m, d_in), lambda j: (0, 0)),
            pl.BlockSpec((d_in, KTILE), win_idx),
            pl.BlockSpec((d_hid, CHUNK), wout_idx),
        ],
        out_specs=pl.BlockSpec((m, d_out), lambda j: (0, 0)),
        scratch_shapes=[
            pltpu.VMEM((m, d_hid), jnp.float32),
            pltpu.VMEM((16, m, HALF), jnp.float32),
            pltpu.SemaphoreType.DMA((16,)),
            pltpu.SemaphoreType.DMA((16,)),
        ],
        compiler_params=pltpu.CompilerParams(
            dimension_semantics=("arbitrary",),
            collective_id=cid,
        ),
    )(x, win, wout)


def kernel(x, Win0, Wout0, Win1, Wout1, Win2, Wout2):
    x = _layer(x, Win0, Wout0, 0)
    x = _layer(x, Win1, Wout1, 1)
    x = _layer(x, Win2, Wout2, 2)
    return x
